# baseline (speedup 1.0000x reference)
import jax
import jax.numpy as jnp
from jax import lax
from jax.experimental import pallas as pl
from jax.experimental.pallas import tpu as pltpu
from jax.experimental.pallas import tpu_sc as plsc

N = 10000
E = 160000
HIDDEN = 256
HEADS = 8
HEAD_DIM = HIDDEN // HEADS
HH = HEADS // 2
HC = HEAD_DIM * HH

NUM_TILES = 16
SUB = 64
EP = ((E + NUM_TILES * 4 * SUB - 1) // (NUM_TILES * 4 * SUB)) * (NUM_TILES * 4 * SUB)
PAD_ROW = N
NPAD = 10240
ROWS_PER_TILE = NPAD // NUM_TILES

EPT_A = EP // 32
NSUB_A = EPT_A // SUB
EPT_B = EP // 16
NSUB_B = EPT_B // SUB


def _proj_body(h_ref, w_ref, b_ref, qk_ref, v_ref):
    acc = (
        jnp.dot(h_ref[...], w_ref[0], preferred_element_type=jnp.float32)
        + b_ref[0, 0:1, :]
    )
    qk_ref[0] = acc[:, :HIDDEN].astype(jnp.bfloat16)
    v_ref[0] = acc[:, HIDDEN:].astype(jnp.bfloat16)


def _project(h, w2, b2):
    blk = 400
    wid = HIDDEN + HC
    grid = (N // blk, 2)
    return pl.pallas_call(
        _proj_body,
        grid=grid,
        in_specs=[
            pl.BlockSpec((blk, HIDDEN), lambda i, j: (i, 0)),
            pl.BlockSpec((1, HIDDEN, wid), lambda i, j: (j, 0, 0)),
            pl.BlockSpec((1, 8, wid), lambda i, j: (j, 0, 0)),
        ],
        out_specs=[
            pl.BlockSpec((1, blk, HIDDEN), lambda i, j: (j, i, 0)),
            pl.BlockSpec((1, blk, HC), lambda i, j: (j, i, 0)),
        ],
        out_shape=[
            jax.ShapeDtypeStruct((2, N, HIDDEN), jnp.bfloat16),
            jax.ShapeDtypeStruct((2, N, HC), jnp.bfloat16),
        ],
    )(h, w2, b2)


def _sc_a_body(qk2, rowp, colp, evp, s_hbm, ex_hbm,
               s_sh,
               rowb0, rowb1, rowb2, rowb3,
               colb0, colb1, colb2, colb3,
               evb0, evb1, evb2, evb3,
               gix0, gix1, gbuf0, gbuf1,
               ex0, ex1,
               semi, semq0, semq1, semx0, semx1, sems0, sems1):
    c = lax.axis_index("c")
    s = lax.axis_index("s")
    tb = c * (EP // 2) + s * EPT_A

    rowb = (rowb0, rowb1, rowb2, rowb3)
    colb = (colb0, colb1, colb2, colb3)
    evb = (evb0, evb1, evb2, evb3)
    gix = (gix0, gix1)
    gbuf = (gbuf0, gbuf1)
    ex2 = (ex0, ex1)
    semq = (semq0, semq1)
    semx = (semx0, semx1)
    sems = (sems0, sems1)

    lane = lax.iota(jnp.int32, 16)
    rot8 = lane ^ 8
    zeros16 = jnp.zeros((16,), jnp.float32)

    def _zero_ex(i, _):
        ex0[i, :] = zeros16
        return 0
    lax.fori_loop(0, SUB, _zero_ex, 0)
    for z in range(ROWS_PER_TILE // SUB):
        zb = s * ROWS_PER_TILE + z * SUB
        pltpu.sync_copy(ex0, s_sh.at[pl.ds(zb, SUB)])
    plsc.subcore_barrier()

    def _mkix2(dst, src_a, src_b):
        def body(j, _):
            dst[pl.ds(j * 16, 16)] = src_a[pl.ds(j * 16, 16)]
            dst[pl.ds(SUB + j * 16, 16)] = src_b[pl.ds(j * 16, 16)] + N
            return 0
        lax.fori_loop(0, SUB // 16, body, 0)

    def _load_idx_sync(slot, i):
        base = tb + i * SUB
        pltpu.sync_copy(rowp.at[pl.ds(base, SUB)], rowb[slot])
        pltpu.sync_copy(colp.at[pl.ds(base, SUB)], colb[slot])
        pltpu.sync_copy(evp.at[pl.ds(base, SUB)], evb[slot])

    def _issue_idx(slot, i):
        base = tb + i * SUB
        pltpu.async_copy(rowp.at[pl.ds(base, SUB)], rowb[slot], semi)
        pltpu.async_copy(colp.at[pl.ds(base, SUB)], colb[slot], semi)
        pltpu.async_copy(evp.at[pl.ds(base, SUB)], evb[slot], semi)

    def _wait_idx():
        for _ in range(3):
            pltpu.make_async_copy(rowp.at[pl.ds(0, SUB)], rowb0, semi).wait()

    LIM_B = [(NSUB_A - 2 - q) // 4 + 1 for q in range(4)]
    LIM_C = [(NSUB_A - 3 - q) // 4 + 1 for q in range(4)]
    NQ = NSUB_A // 4

    def _when_lim(m, lim):
        if lim >= NQ:
            return pl.when(m >= 0)
        return pl.when(m < lim)

    def _step(m, q):
        i = 4 * m + q
        p = q % 2
        slot_i = q
        slot_n = (q + 1) % 4

        def _drain():
            pltpu.make_async_copy(ex2[p], ex_hbm.at[pl.ds(0, SUB)],
                                  semx[p]).wait()
            pltpu.make_async_copy(ex2[p], s_sh.at[pl.ds(0, SUB)],
                                  sems[p]).wait()
        if q < 2:
            pl.when(m > 0)(_drain)
        else:
            _drain()

        @_when_lim(m, LIM_B[q])
        def _():
            _wait_idx()
            _mkix2(gix[1 - p], rowb[slot_n], colb[slot_n])
            pltpu.async_copy(qk2.at[gix[1 - p]], gbuf[1 - p], semq[1 - p])

        @_when_lim(m, LIM_C[q])
        def _():
            _issue_idx((q + 2) % 4, i + 2)

        pltpu.make_async_copy(qk2.at[gix[p]], gbuf[p], semq[p]).wait()

        exd = ex2[p]
        qrp = gbuf[p]
        evd = evb[slot_i]

        def _edge(ed, _):
            acc = zeros16
            for t in range(HIDDEN // 32):
                u0q, u1q = plsc.unpack(qrp[ed, pl.ds(t * 32, 32)],
                                       format=plsc.PackFormat.INTERLEAVED)
                u0k, u1k = plsc.unpack(qrp[SUB + ed, pl.ds(t * 32, 32)],
                                       format=plsc.PackFormat.INTERLEAVED)
                acc = acc + u0q * u0k + u1q * u1k
            acc = acc + acc.at[rot8].get(mode="promise_in_bounds")
            grp = evd[pl.ds((ed // 16) * 16, 16)]
            ev = grp.at[jnp.full((16,), ed % 16, jnp.int32)].get(
                mode="promise_in_bounds")
            exd[ed, :] = jnp.exp(acc * ev)
            return 0
        lax.fori_loop(0, SUB, _edge, 0)

        pltpu.async_copy(ex2[p], s_sh.at[rowb[slot_i]], sems[p], add=True)
        pltpu.async_copy(ex2[p], ex_hbm.at[pl.ds(tb + i * SUB, SUB)], semx[p])
        return 0

    _load_idx_sync(0, 0)
    _mkix2(gix[0], rowb[0], colb[0])
    pltpu.async_copy(qk2.at[gix[0]], gbuf[0], semq[0])
    _issue_idx(1, 1)

    def _quad(m, _):
        for q in range(4):
            _step(m, q)
        return 0
    lax.fori_loop(0, NQ, _quad, 0)

    for p in range(2):
        pltpu.make_async_copy(ex2[p], ex_hbm.at[pl.ds(0, SUB)], semx[p]).wait()
        pltpu.make_async_copy(ex2[p], s_sh.at[pl.ds(0, SUB)], sems[p]).wait()
    plsc.subcore_barrier()

    for z in range(ROWS_PER_TILE // SUB):
        zb = s * ROWS_PER_TILE + z * SUB
        pltpu.sync_copy(s_sh.at[pl.ds(zb, SUB)],
                        s_hbm.at[pl.ds(c * NPAD + zb, SUB)])


def _sddmm_exp(qk_flat, rowp, colp, evp):
    mesh = plsc.VectorSubcoreMesh(core_axis_name="c", subcore_axis_name="s")
    fn = pl.kernel(
        _sc_a_body,
        out_type=[
            jax.ShapeDtypeStruct((2 * NPAD, 16), jnp.float32),
            jax.ShapeDtypeStruct((EP, 16), jnp.float32),
        ],
        mesh=mesh,
        compiler_params=pltpu.CompilerParams(use_tc_tiling_on_sc=False,
                                             needs_layout_passes=False),
        scratch_types=[
            pltpu.VMEM_SHARED((NPAD, 16), jnp.float32),
            pltpu.VMEM((SUB,), jnp.int32),
            pltpu.VMEM((SUB,), jnp.int32),
            pltpu.VMEM((SUB,), jnp.int32),
            pltpu.VMEM((SUB,), jnp.int32),
            pltpu.VMEM((SUB,), jnp.int32),
            pltpu.VMEM((SUB,), jnp.int32),
            pltpu.VMEM((SUB,), jnp.int32),
            pltpu.VMEM((SUB,), jnp.int32),
            pltpu.VMEM((SUB,), jnp.float32),
            pltpu.VMEM((SUB,), jnp.float32),
            pltpu.VMEM((SUB,), jnp.float32),
            pltpu.VMEM((SUB,), jnp.float32),
            pltpu.VMEM((2 * SUB,), jnp.int32),
            pltpu.VMEM((2 * SUB,), jnp.int32),
            pltpu.VMEM((2 * SUB, HIDDEN), jnp.bfloat16),
            pltpu.VMEM((2 * SUB, HIDDEN), jnp.bfloat16),
            pltpu.VMEM((SUB, 16), jnp.float32),
            pltpu.VMEM((SUB, 16), jnp.float32),
            pltpu.SemaphoreType.DMA,
            pltpu.SemaphoreType.DMA,
            pltpu.SemaphoreType.DMA,
            pltpu.SemaphoreType.DMA,
            pltpu.SemaphoreType.DMA,
            pltpu.SemaphoreType.DMA,
            pltpu.SemaphoreType.DMA,
        ],
    )
    return fn(qk_flat, rowp, colp, evp)


def _sc_b_body(v2, rowp, colp, ex_hbm, s_hbm, out_hbm,
               out_sh,
               rowb0, rowb1, rowb2, rowb3,
               colb0, colb1, colb2, colb3,
               vix0, vix1, vbuf0, vbuf1,
               wv0, wv1, ex0, ex1, sb0, sb1,
               semi, semq0, semq1, semx0, semx1, sems0, sems1):
    c = lax.axis_index("c")
    s = lax.axis_index("s")
    tb = s * EPT_B

    rowb = (rowb0, rowb1, rowb2, rowb3)
    colb = (colb0, colb1, colb2, colb3)
    vix = (vix0, vix1)
    vbuf = (vbuf0, vbuf1)
    wv = (wv0, wv1)
    ex2 = (ex0, ex1)
    semq = (semq0, semq1)
    semx = (semx0, semx1)
    sems = (sems0, sems1)

    lane = lax.iota(jnp.int32, 16)
    permc = (lane & 3) + 4 * c
    zeros16 = jnp.zeros((16,), jnp.float32)
    voff = c * N

    def _zero_wv(i, _):
        for j in range(HC // 16):
            wv0[i, pl.ds(j * 16, 16)] = zeros16
        return 0
    lax.fori_loop(0, SUB, _zero_wv, 0)
    for z in range(ROWS_PER_TILE // SUB):
        zb = s * ROWS_PER_TILE + z * SUB
        pltpu.sync_copy(wv0, out_sh.at[pl.ds(zb, SUB)])
    plsc.subcore_barrier()

    def _mkix(dst, src):
        def body(j, _):
            dst[pl.ds(j * 16, 16)] = src[pl.ds(j * 16, 16)] + voff
            return 0
        lax.fori_loop(0, SUB // 16, body, 0)

    def _load_idx_sync(slot, i):
        base = tb + i * SUB
        pltpu.sync_copy(rowp.at[pl.ds(base, SUB)], rowb[slot])
        pltpu.sync_copy(colp.at[pl.ds(base, SUB)], colb[slot])

    def _issue_idx(slot, i):
        base = tb + i * SUB
        pltpu.async_copy(rowp.at[pl.ds(base, SUB)], rowb[slot], semi)
        pltpu.async_copy(colp.at[pl.ds(base, SUB)], colb[slot], semi)

    def _wait_idx():
        for _ in range(2):
            pltpu.make_async_copy(rowp.at[pl.ds(0, SUB)], rowb0, semi).wait()

    LIM_B = [(NSUB_B - 2 - q) // 4 + 1 for q in range(4)]
    LIM_C = [(NSUB_B - 3 - q) // 4 + 1 for q in range(4)]
    NQ = NSUB_B // 4

    def _when_lim(m, lim):
        if lim >= NQ:
            return pl.when(m >= 0)
        return pl.when(m < lim)

    def _step(m, q):
        i = 4 * m + q
        p = q % 2
        slot_i = q
        slot_n = (q + 1) % 4

        def _drain():
            pltpu.make_async_copy(wv[p], out_sh.at[pl.ds(0, SUB)],
                                  sems[p]).wait()
        if q < 2:
            pl.when(m > 0)(_drain)
        else:
            _drain()

        @_when_lim(m, LIM_B[q])
        def _():
            _wait_idx()
            _mkix(vix[1 - p], colb[slot_n])
            pltpu.async_copy(v2.at[vix[1 - p]], vbuf[1 - p], semq[1 - p])
            pltpu.async_copy(ex_hbm.at[pl.ds(tb + (i + 1) * SUB, SUB)],
                             ex2[1 - p], semx[1 - p])

        @_when_lim(m, LIM_C[q])
        def _():
            _issue_idx((q + 2) % 4, i + 2)

        pltpu.make_async_copy(v2.at[vix[p]], vbuf[p], semq[p]).wait()
        pltpu.make_async_copy(ex_hbm.at[pl.ds(0, SUB)], ex2[p], semx[p]).wait()

        vrp = vbuf[p]
        wvp = wv[p]
        exd = ex2[p]

        def _edge(ed, _):
            w = exd[ed, :].at[permc].get(mode="promise_in_bounds")
            for t in range(HC // 32):
                u0, u1 = plsc.unpack(vrp[ed, pl.ds(t * 32, 32)],
                                     format=plsc.PackFormat.INTERLEAVED)
                wvp[ed, pl.ds(t * 16, 16)] = w * u0
                wvp[ed, pl.ds(HC // 2 + t * 16, 16)] = w * u1
            return 0
        lax.fori_loop(0, SUB, _edge, 0)

        pltpu.async_copy(wv[p], out_sh.at[rowb[slot_i]], sems[p], add=True)
        return 0

    _load_idx_sync(0, 0)
    _mkix(vix[0], colb[0])
    pltpu.async_copy(v2.at[vix[0]], vbuf[0], semq[0])
    pltpu.async_copy(ex_hbm.at[pl.ds(tb, SUB)], ex2[0], semx[0])
    _issue_idx(1, 1)

    def _quad(m, _):
        for q in range(4):
            _step(m, q)
        return 0
    lax.fori_loop(0, NQ, _quad, 0)

    for p in range(2):
        pltpu.make_async_copy(wv[p], out_sh.at[pl.ds(0, SUB)], sems[p]).wait()
    plsc.subcore_barrier()

    for z in range(ROWS_PER_TILE // SUB):
        zb = s * ROWS_PER_TILE + z * SUB
        pltpu.sync_copy(out_sh.at[pl.ds(zb, SUB)], wv0)
        pltpu.sync_copy(s_hbm.at[pl.ds(zb, SUB)], sb0)
        pltpu.sync_copy(s_hbm.at[pl.ds(NPAD + zb, SUB)], sb1)

        def _norm(r, _):
            srow = sb0[r, :] + sb1[r, :]
            sd = jnp.maximum(srow.at[permc].get(mode="promise_in_bounds"),
                             1e-20)
            for t in range(HC // 16):
                wv0[r, pl.ds(t * 16, 16)] = wv0[r, pl.ds(t * 16, 16)] / sd
            return 0
        lax.fori_loop(0, SUB, _norm, 0)

        pltpu.sync_copy(wv0, out_hbm.at[pl.ds(c * NPAD + zb, SUB)])


def _spmm(v_flat, rowp, colp, ex, s_part):
    mesh = plsc.VectorSubcoreMesh(core_axis_name="c", subcore_axis_name="s")
    fn = pl.kernel(
        _sc_b_body,
        out_type=jax.ShapeDtypeStruct((2 * NPAD, HC), jnp.float32),
        mesh=mesh,
        compiler_params=pltpu.CompilerParams(use_tc_tiling_on_sc=False,
                                             needs_layout_passes=False),
        scratch_types=[
            pltpu.VMEM_SHARED((NPAD, HC), jnp.float32),
            pltpu.VMEM((SUB,), jnp.int32),
            pltpu.VMEM((SUB,), jnp.int32),
            pltpu.VMEM((SUB,), jnp.int32),
            pltpu.VMEM((SUB,), jnp.int32),
            pltpu.VMEM((SUB,), jnp.int32),
            pltpu.VMEM((SUB,), jnp.int32),
            pltpu.VMEM((SUB,), jnp.int32),
            pltpu.VMEM((SUB,), jnp.int32),
            pltpu.VMEM((SUB,), jnp.int32),
            pltpu.VMEM((SUB,), jnp.int32),
            pltpu.VMEM((SUB, HC), jnp.bfloat16),
            pltpu.VMEM((SUB, HC), jnp.bfloat16),
            pltpu.VMEM((SUB, HC), jnp.float32),
            pltpu.VMEM((SUB, HC), jnp.float32),
            pltpu.VMEM((SUB, 16), jnp.float32),
            pltpu.VMEM((SUB, 16), jnp.float32),
            pltpu.VMEM((SUB, 16), jnp.float32),
            pltpu.VMEM((SUB, 16), jnp.float32),
            pltpu.SemaphoreType.DMA,
            pltpu.SemaphoreType.DMA,
            pltpu.SemaphoreType.DMA,
            pltpu.SemaphoreType.DMA,
            pltpu.SemaphoreType.DMA,
            pltpu.SemaphoreType.DMA,
            pltpu.SemaphoreType.DMA,
        ],
    )
    return fn(v_flat, rowp, colp, ex, s_part)


def kernel(h, edge_index, edge_val, Wq, bq, Wk, bk, Wv, bv):
    scaling = HEAD_DIM ** (-0.5)

    pq = jnp.arange(HIDDEN, dtype=jnp.int32)
    colq = (2 * (pq // 16) + pq % 2) * HEADS + (pq // 2) % HEADS
    pv = jnp.arange(HC, dtype=jnp.int32)
    col_a = (2 * (pv // 8) + pv % 2) * HEADS + (pv // 2) % HH
    col_b = col_a + HH

    wq_s = Wq * scaling
    bq_s = bq * scaling
    w2 = jnp.stack([
        jnp.concatenate([wq_s[colq].T, Wv[col_a].T], axis=1),
        jnp.concatenate([Wk[colq].T, Wv[col_b].T], axis=1),
    ])
    b2 = jnp.stack([
        jnp.concatenate([bq_s[colq], bv[col_a]]),
        jnp.concatenate([bk[colq], bv[col_b]]),
    ])
    b2 = jnp.broadcast_to(b2[:, None, :], (2, 8, HIDDEN + HC))

    qk, v2 = _project(h, w2, b2)
    qk_flat = qk.reshape(2 * N, HIDDEN)
    v_flat = v2.reshape(2 * N, HC)

    row = edge_index[0]
    col = edge_index[1]
    pad = EP - E
    rowp = jnp.concatenate([row, jnp.full((pad,), PAD_ROW, jnp.int32)])
    colp = jnp.concatenate([col, jnp.zeros((pad,), jnp.int32)])
    evp = jnp.concatenate([edge_val, jnp.zeros((pad,), jnp.float32)])

    s_part, ex = _sddmm_exp(qk_flat, rowp, colp, evp)
    outcat = _spmm(v_flat, rowp, colp, ex, s_part)

    both = jnp.concatenate(
        [outcat[:N, :HC // 2], outcat[:N, HC // 2:],
         outcat[NPAD:NPAD + N, :HC // 2], outcat[NPAD:NPAD + N, HC // 2:]],
        axis=1)
    p = jnp.arange(HIDDEN, dtype=jnp.int32)
    hh = p % HEADS
    dd = p // HEADS
    inv = ((hh >= HH) * 2 + dd % 2) * (HC // 2) + (dd // 2) * HH + hh % HH
    return both[:, inv]


# speedup vs baseline: 1.1345x; 1.1345x over previous
import jax
import jax.numpy as jnp
from jax import lax
from jax.experimental import pallas as pl
from jax.experimental.pallas import tpu as pltpu
from jax.experimental.pallas import tpu_sc as plsc

N = 10000
E = 160000
HIDDEN = 256
HEADS = 8
HEAD_DIM = HIDDEN // HEADS
HH = HEADS // 2
HC = HEAD_DIM * HH

NUM_TILES = 16
SUB = 64
EP = ((E + NUM_TILES * 4 * SUB - 1) // (NUM_TILES * 4 * SUB)) * (NUM_TILES * 4 * SUB)
EDGES_PER_TILE = EP // NUM_TILES
NSUB = EDGES_PER_TILE // SUB
NQUAD = NSUB // 4
PAD_ROW = N
NPAD = 10240
ROWS_PER_TILE = NPAD // NUM_TILES


def _proj_body(h_ref, w_ref, b_ref, out_ref):
    out_ref[0] = (
        jnp.dot(h_ref[...], w_ref[0], preferred_element_type=jnp.float32)
        + b_ref[0, 0:1, :]
    ).astype(jnp.bfloat16)


def _project(h, w6, b6):
    blk = 400
    grid = (N // blk, 6)
    return pl.pallas_call(
        _proj_body,
        grid=grid,
        in_specs=[
            pl.BlockSpec((blk, HIDDEN), lambda i, j: (i, 0)),
            pl.BlockSpec((1, HIDDEN, HC), lambda i, j: (j, 0, 0)),
            pl.BlockSpec((1, 8, HC), lambda i, j: (j, 0, 0)),
        ],
        out_specs=pl.BlockSpec((1, blk, HC), lambda i, j: (j, i, 0)),
        out_shape=jax.ShapeDtypeStruct((6, N, HC), jnp.bfloat16),
    )(h, w6, b6)


def _sc_body(qkv, rowp, colp, evp, out_e_hbm, out_o_hbm, ex_hbm,
             s_sh, out_e_sh, out_o_sh,
             rowb0, rowb1, rowb2, rowb3,
             colb0, colb1, colb2, colb3,
             evb0, evb1, evb2, evb3,
             qix0, qix1, gix0, gix1,
             gbuf0, gbuf1,
             wve0, wve1, wvo0, wvo1,
             ex0, ex1, sc0, sc1,
             semi, semq0, semq1, semk0, semk1,
             semx0, semx1, sems0, sems1):
    c = lax.axis_index("c")
    s = lax.axis_index("s")
    tb = s * EDGES_PER_TILE

    rowb = (rowb0, rowb1, rowb2, rowb3)
    colb = (colb0, colb1, colb2, colb3)
    evb = (evb0, evb1, evb2, evb3)
    qix = (qix0, qix1)
    gix = (gix0, gix1)
    gbuf = (gbuf0, gbuf1)
    wve = (wve0, wve1)
    wvo = (wvo0, wvo1)
    ex2 = (ex0, ex1)
    sc2 = (sc0, sc1)
    semq = (semq0, semq1)
    semk = (semk0, semk1)
    semx = (semx0, semx1)
    sems = (sems0, sems1)

    lane = lax.iota(jnp.int32, 16)
    rot8 = lane ^ 8
    rot4 = lane ^ 4
    zeros16 = jnp.zeros((16,), jnp.float32)

    def _zero_wv(i, _):
        for j in range(64 // 16):
            wve0[i, pl.ds(j * 16, 16)] = zeros16
        return 0
    lax.fori_loop(0, SUB, _zero_wv, 0)

    def _zero_ex(i, _):
        ex0[i, :] = zeros16
        return 0
    lax.fori_loop(0, SUB, _zero_ex, 0)

    for z in range(ROWS_PER_TILE // SUB):
        zb = s * ROWS_PER_TILE + z * SUB
        pltpu.sync_copy(wve0, out_e_sh.at[pl.ds(zb, SUB)])
        pltpu.sync_copy(wve0, out_o_sh.at[pl.ds(zb, SUB)])
        pltpu.sync_copy(ex0, s_sh.at[pl.ds(zb, SUB)])
    plsc.subcore_barrier()

    qoff = c * N
    koff = (2 + c) * N
    voff = (4 + c) * N
    exbase = c * EP + tb

    def _mkix(dst, src, off):
        def body(j, _):
            dst[pl.ds(j * 16, 16)] = src[pl.ds(j * 16, 16)] + off
            return 0
        lax.fori_loop(0, SUB // 16, body, 0)

    def _mkix2(dst, src_a, off_a, src_b, off_b):
        def body(j, _):
            dst[pl.ds(j * 16, 16)] = src_a[pl.ds(j * 16, 16)] + off_a
            dst[pl.ds(SUB + j * 16, 16)] = src_b[pl.ds(j * 16, 16)] + off_b
            return 0
        lax.fori_loop(0, SUB // 16, body, 0)

    def _load_idx_sync(slot, i):
        base = tb + i * SUB
        pltpu.sync_copy(rowp.at[pl.ds(base, SUB)], rowb[slot])
        pltpu.sync_copy(colp.at[pl.ds(base, SUB)], colb[slot])
        pltpu.sync_copy(evp.at[pl.ds(base, SUB)], evb[slot])

    def _issue_idx(slot, i, with_ev):
        base = tb + i * SUB
        pltpu.async_copy(rowp.at[pl.ds(base, SUB)], rowb[slot], semi)
        pltpu.async_copy(colp.at[pl.ds(base, SUB)], colb[slot], semi)
        if with_ev:
            pltpu.async_copy(evp.at[pl.ds(base, SUB)], evb[slot], semi)

    def _wait_idx(with_ev):
        n = 3 if with_ev else 2
        for _ in range(n):
            pltpu.make_async_copy(rowp.at[pl.ds(0, SUB)], rowb0, semi).wait()

    LIM_B = [(NSUB - 2 - q) // 4 + 1 for q in range(4)]
    LIM_C = [(NSUB - 3 - q) // 4 + 1 for q in range(4)]

    def _when_lim(m, lim):
        if lim >= NQUAD:
            return pl.when(m >= 0)
        return pl.when(m < lim)

    def _pass_a_step(m, q):
        i = 4 * m + q
        p = q % 2
        slot_i = q
        slot_n = (q + 1) % 4

        def _drain():
            pltpu.make_async_copy(ex2[p], ex_hbm.at[pl.ds(exbase, SUB)],
                                  semx[p]).wait()
            pltpu.make_async_copy(ex2[p], s_sh.at[pl.ds(0, SUB)],
                                  sems[p]).wait()
        if q < 2:
            pl.when(m > 0)(_drain)
        else:
            _drain()

        @_when_lim(m, LIM_B[q])
        def _():
            _wait_idx(True)
            _mkix2(gix[1 - p], rowb[slot_n], qoff, colb[slot_n], koff)
            pltpu.async_copy(qkv.at[gix[1 - p]], gbuf[1 - p], semq[1 - p])

        @_when_lim(m, LIM_C[q])
        def _():
            _issue_idx((q + 2) % 4, i + 2, True)

        pltpu.make_async_copy(qkv.at[gix[p]], gbuf[p], semq[p]).wait()

        exd = ex2[p]
        qrp = gbuf[p]
        evd = evb[slot_i]

        def _edge(ed, _):
            acc = zeros16
            for t in range(HC // 32):
                u0q, u1q = plsc.unpack(qrp[ed, pl.ds(t * 32, 32)],
                                       format=plsc.PackFormat.INTERLEAVED)
                u0k, u1k = plsc.unpack(qrp[SUB + ed, pl.ds(t * 32, 32)],
                                       format=plsc.PackFormat.INTERLEAVED)
                acc = acc + u0q * u0k + u1q * u1k
            acc = acc + acc.at[rot8].get(mode="promise_in_bounds")
            acc = acc + acc.at[rot4].get(mode="promise_in_bounds")
            grp = evd[pl.ds((ed // 16) * 16, 16)]
            ev = grp.at[jnp.full((16,), ed % 16, jnp.int32)].get(
                mode="promise_in_bounds")
            exd[ed, :] = jnp.exp(acc * ev)
            return 0
        lax.fori_loop(0, SUB, _edge, 0)

        pltpu.async_copy(ex2[p], s_sh.at[rowb[slot_i]], sems[p], add=True)
        pltpu.async_copy(ex2[p], ex_hbm.at[pl.ds(exbase + i * SUB, SUB)],
                         semx[p])
        return 0

    _load_idx_sync(0, 0)
    _mkix2(gix[0], rowb[0], qoff, colb[0], koff)
    pltpu.async_copy(qkv.at[gix[0]], gbuf[0], semq[0])
    _issue_idx(1, 1, True)

    def _quad_a(m, _):
        for q in range(4):
            _pass_a_step(m, q)
        return 0
    lax.fori_loop(0, NQUAD, _quad_a, 0)

    for p in range(2):
        pltpu.make_async_copy(ex2[p], ex_hbm.at[pl.ds(exbase, SUB)],
                              semx[p]).wait()
        pltpu.make_async_copy(ex2[p], s_sh.at[pl.ds(0, SUB)], sems[p]).wait()

    plsc.subcore_barrier()

    def _pass_b_step(m, q):
        i = 4 * m + q
        p = q % 2
        slot_i = q
        slot_n = (q + 1) % 4

        def _drain():
            pltpu.make_async_copy(wve[p], out_e_sh.at[pl.ds(0, SUB)],
                                  sems[p]).wait()
            pltpu.make_async_copy(wvo[p], out_o_sh.at[pl.ds(0, SUB)],
                                  sems[p]).wait()
        if q < 2:
            pl.when(m > 0)(_drain)
        else:
            _drain()

        @_when_lim(m, LIM_B[q])
        def _():
            _wait_idx(False)
            _mkix(qix[1 - p], colb[slot_n], voff)
            pltpu.async_copy(qkv.at[qix[1 - p]],
                             gbuf[1 - p].at[pl.ds(0, SUB)], semq[1 - p])
            pltpu.async_copy(s_sh.at[rowb[slot_n]], sc2[1 - p], semk[1 - p])
            pltpu.async_copy(ex_hbm.at[pl.ds(exbase + (i + 1) * SUB, SUB)],
                             ex2[1 - p], semx[1 - p])

        @_when_lim(m, LIM_C[q])
        def _():
            _issue_idx((q + 2) % 4, i + 2, False)

        pltpu.make_async_copy(qkv.at[qix[p]],
                              gbuf[p].at[pl.ds(0, SUB)], semq[p]).wait()
        pltpu.make_async_copy(s_sh.at[rowb[slot_i]], sc2[p], semk[p]).wait()
        pltpu.make_async_copy(ex_hbm.at[pl.ds(0, SUB)], ex2[p], semx[p]).wait()

        vrp = gbuf[p]
        wep = wve[p]
        wop = wvo[p]
        exd = ex2[p]
        scd = sc2[p]

        def _edge(ed, _):
            w = exd[ed, :] / scd[ed, :]
            for t in range(HC // 32):
                u0, u1 = plsc.unpack(vrp[ed, pl.ds(t * 32, 32)],
                                     format=plsc.PackFormat.INTERLEAVED)
                wep[ed, pl.ds(t * 16, 16)] = w * u0
                wop[ed, pl.ds(t * 16, 16)] = w * u1
            return 0
        lax.fori_loop(0, SUB, _edge, 0)

        pltpu.async_copy(wve[p], out_e_sh.at[rowb[slot_i]], sems[p], add=True)
        pltpu.async_copy(wvo[p], out_o_sh.at[rowb[slot_i]], sems[p], add=True)
        return 0

    _load_idx_sync(0, 0)
    _mkix(qix[0], colb[0], voff)
    pltpu.async_copy(qkv.at[qix[0]], gbuf[0].at[pl.ds(0, SUB)], semq[0])
    pltpu.async_copy(s_sh.at[rowb[0]], sc2[0], semk[0])
    pltpu.async_copy(ex_hbm.at[pl.ds(exbase, SUB)], ex2[0], semx[0])
    _issue_idx(1, 1, False)

    def _quad_b(m, _):
        for q in range(4):
            _pass_b_step(m, q)
        return 0
    lax.fori_loop(0, NQUAD, _quad_b, 0)

    for p in range(2):
        pltpu.make_async_copy(wve[p], out_e_sh.at[pl.ds(0, SUB)],
                              sems[p]).wait()
        pltpu.make_async_copy(wvo[p], out_o_sh.at[pl.ds(0, SUB)],
                              sems[p]).wait()

    plsc.subcore_barrier()

    for z in range(ROWS_PER_TILE // SUB):
        zb = s * ROWS_PER_TILE + z * SUB
        pltpu.sync_copy(out_e_sh.at[pl.ds(zb, SUB)],
                        out_e_hbm.at[pl.ds(c * NPAD + zb, SUB)])
        pltpu.sync_copy(out_o_sh.at[pl.ds(zb, SUB)],
                        out_o_hbm.at[pl.ds(c * NPAD + zb, SUB)])


def _sparse_attention(qkv_flat, rowp, colp, evp):
    mesh = plsc.VectorSubcoreMesh(core_axis_name="c", subcore_axis_name="s")
    fn = pl.kernel(
        _sc_body,
        out_type=[
            jax.ShapeDtypeStruct((2 * NPAD, HC // 2), jnp.float32),
            jax.ShapeDtypeStruct((2 * NPAD, HC // 2), jnp.float32),
            jax.ShapeDtypeStruct((2 * EP, 16), jnp.float32),
        ],
        mesh=mesh,
        compiler_params=pltpu.CompilerParams(use_tc_tiling_on_sc=False,
                                             needs_layout_passes=False),
        scratch_types=[
            pltpu.VMEM_SHARED((NPAD, 16), jnp.float32),
            pltpu.VMEM_SHARED((NPAD, HC // 2), jnp.float32),
            pltpu.VMEM_SHARED((NPAD, HC // 2), jnp.float32),
            pltpu.VMEM((SUB,), jnp.int32),
            pltpu.VMEM((SUB,), jnp.int32),
            pltpu.VMEM((SUB,), jnp.int32),
            pltpu.VMEM((SUB,), jnp.int32),
            pltpu.VMEM((SUB,), jnp.int32),
            pltpu.VMEM((SUB,), jnp.int32),
            pltpu.VMEM((SUB,), jnp.int32),
            pltpu.VMEM((SUB,), jnp.int32),
            pltpu.VMEM((SUB,), jnp.float32),
            pltpu.VMEM((SUB,), jnp.float32),
            pltpu.VMEM((SUB,), jnp.float32),
            pltpu.VMEM((SUB,), jnp.float32),
            pltpu.VMEM((SUB,), jnp.int32),
            pltpu.VMEM((SUB,), jnp.int32),
            pltpu.VMEM((2 * SUB,), jnp.int32),
            pltpu.VMEM((2 * SUB,), jnp.int32),
            pltpu.VMEM((2 * SUB, HC), jnp.bfloat16),
            pltpu.VMEM((2 * SUB, HC), jnp.bfloat16),
            pltpu.VMEM((SUB, HC // 2), jnp.float32),
            pltpu.VMEM((SUB, HC // 2), jnp.float32),
            pltpu.VMEM((SUB, HC // 2), jnp.float32),
            pltpu.VMEM((SUB, HC // 2), jnp.float32),
            pltpu.VMEM((SUB, 16), jnp.float32),
            pltpu.VMEM((SUB, 16), jnp.float32),
            pltpu.VMEM((SUB, 16), jnp.float32),
            pltpu.VMEM((SUB, 16), jnp.float32),
            pltpu.SemaphoreType.DMA,
            pltpu.SemaphoreType.DMA,
            pltpu.SemaphoreType.DMA,
            pltpu.SemaphoreType.DMA,
            pltpu.SemaphoreType.DMA,
            pltpu.SemaphoreType.DMA,
            pltpu.SemaphoreType.DMA,
            pltpu.SemaphoreType.DMA,
            pltpu.SemaphoreType.DMA,
        ],
    )
    out_e, out_o, _ex = fn(qkv_flat, rowp, colp, evp)
    return out_e, out_o


def kernel(h, edge_index, edge_val, Wq, bq, Wk, bk, Wv, bv):
    scaling = HEAD_DIM ** (-0.5)

    pp = jnp.arange(HC, dtype=jnp.int32)
    hp = (pp // 2) % HH
    dp = 2 * (pp // 8) + (pp % 2)
    col_a = dp * HEADS + hp
    col_b = col_a + HH

    wq_s = Wq * scaling
    bq_s = bq * scaling
    w6 = jnp.stack([
        wq_s[col_a].T, wq_s[col_b].T,
        Wk[col_a].T, Wk[col_b].T,
        Wv[col_a].T, Wv[col_b].T,
    ])
    b6 = jnp.stack([
        bq_s[col_a], bq_s[col_b],
        bk[col_a], bk[col_b],
        bv[col_a], bv[col_b],
    ])
    b6 = jnp.broadcast_to(b6[:, None, :], (6, 8, HC))

    qkv = _project(h, w6, b6)
    qkv_flat = qkv.reshape(6 * N, HC)

    row = edge_index[0]
    col = edge_index[1]
    pad = EP - E
    rowp = jnp.concatenate([row, jnp.full((pad,), PAD_ROW, jnp.int32)])
    colp = jnp.concatenate([col, jnp.zeros((pad,), jnp.int32)])
    evp = jnp.concatenate([edge_val, jnp.zeros((pad,), jnp.float32)])

    out_e, out_o = _sparse_attention(qkv_flat, rowp, colp, evp)

    both = jnp.concatenate(
        [out_e[:N], out_o[:N], out_e[NPAD:NPAD + N], out_o[NPAD:NPAD + N]],
        axis=1)
    p = jnp.arange(HIDDEN, dtype=jnp.int32)
    hh = p % HEADS
    dd = p // HEADS
    inv = ((hh >= HH) * 2 + dd % 2) * (HC // 2) + (dd // 2) * HH + hh % HH
    return both[:, inv]
